# baseline (speedup 1.0000x reference)
import jax
import jax.numpy as jnp
from jax import lax
from jax.experimental import pallas as pl
from jax.experimental.pallas import tpu as pltpu
from jax.experimental.pallas import tpu_sc as plsc

_NC = 2
_NS = 16
_NW = _NC * _NS
_CH = 256
_NB = 5
_K = 3

_VP = 1000064


def _relayout_body(tt_hbm, tail_hbm, out_hbm, a_v, b_v, tail_v, sem_i, sem_o):
    nv = tt_hbm.shape[1]
    nblk = nv // 128
    wid = lax.axis_index("s") * _NC + lax.axis_index("c")
    per_w = (nblk + _NW - 1) // _NW
    iota = lax.iota(jnp.int32, 16)

    def blk(t):
        return t * _NW + wid

    def in_src(t):
        return tt_hbm.at[:, pl.ds(pl.multiple_of(blk(t) * 128, 128), 128)]

    def out_dst(t):
        return out_hbm.at[pl.ds(pl.multiple_of(blk(t) * 64, 64), 64)]

    @pl.when(blk(0) < nblk)
    def _prime():
        pltpu.async_copy(in_src(0), a_v.at[0], sem_i.at[0])

    def block(t, carry):
        s = t % 2

        @pl.when(blk(t + 1) < nblk)
        def _prefetch():
            pltpu.async_copy(in_src(t + 1), a_v.at[1 - s], sem_i.at[1 - s])

        @pl.when(blk(t) < nblk)
        def _wait_in():
            pltpu.make_async_copy(in_src(t), a_v.at[s], sem_i.at[s]).wait()

        @pl.when(jnp.logical_and(t >= 2, blk(t - 2) < nblk))
        def _drain():
            pltpu.make_async_copy(b_v.at[s], out_dst(t - 2), sem_o.at[s]).wait()

        plsc.subcore_barrier()

        @pl.when(blk(t) < nblk)
        def _transpose():
            @plsc.parallel_loop(0, 64, unroll=8)
            def _col_pair(vv):
                for par in range(2):
                    v = 2 * vv + par
                    for k in range(4):
                        vals = plsc.load_gather(
                            a_v.at[s],
                            [iota + 16 * k, jnp.full((16,), v, jnp.int32)],
                        )
                        b_v[s, vv, pl.ds(par * 64 + 16 * k, 16)] = vals

        plsc.subcore_barrier()

        @pl.when(blk(t) < nblk)
        def _start_out():
            pltpu.async_copy(b_v.at[s], out_dst(t), sem_o.at[s])

        return carry

    lax.fori_loop(0, per_w, block, 0)

    for d in range(2):
        t = per_w - 2 + d

        @pl.when(jnp.logical_and(t >= 0, blk(t) < nblk))
        def _final_drain():
            pltpu.make_async_copy(b_v.at[t % 2], out_dst(t), sem_o.at[t % 2]).wait()

    @pl.when(wid == 0)
    def _tail():
        pltpu.sync_copy(tail_hbm, tail_v)
        pltpu.sync_copy(tail_v, out_hbm.at[pl.ds((nblk * 128) // 2, tail_hbm.shape[0])])


def _gather_body_old(x_hbm, table_hbm, out_hbm, idx_all, rows_v, sem_g, sem_w):
    n = x_hbm.shape[0]
    per_w = n // _NW
    nchunk = per_w // _CH
    wid = lax.axis_index("s") * _NC + lax.axis_index("c")
    base = wid * per_w

    def idx_slice(i):
        return idx_all.at[pl.ds(i * _CH, _CH)]

    def out_slice(i):
        return out_hbm.at[pl.ds(base + i * _CH, _CH)]

    def chunk(i, carry):
        pltpu.async_copy(table_hbm.at[idx_slice(i)], rows_v.at[0], sem_g.at[0]).wait()
        pltpu.sync_copy(rows_v.at[0], out_slice(i))
        return carry

    pltpu.sync_copy(x_hbm.at[pl.ds(base, per_w)], idx_all)
    lax.fori_loop(0, nchunk, chunk, 0)


def _gather_body(x_hbm, table_hbm, out_hbm, idx_all, idx_eo, rows_v, sem_g, sem_w):
    n = x_hbm.shape[0]
    per_w = n // _NW
    nchunk = per_w // _CH
    half = _CH // 2
    wid = lax.axis_index("s") * _NC + lax.axis_index("c")
    base = wid * per_w
    iota = lax.iota(jnp.int32, 16)

    def split_idx(i, b):
        for e in range(2):
            for m in range(half // 16):
                src = i * _CH + e + 2 * (iota + 16 * m)
                idx_eo[b, e, pl.ds(16 * m, 16)] = plsc.load_gather(idx_all, [src])

    def out_slice(i, e):
        r0 = pl.multiple_of((base + i * _CH) // 2, half)
        return out_hbm.at[pl.ds(r0, half), pl.ds(e * 64, 64)]

    def start_gather(i, b):
        for e in range(2):
            pltpu.async_copy(
                table_hbm.at[idx_eo.at[b, e]], rows_v.at[b, e], sem_g.at[b]
            )

    def wait_gather(i, b):
        for e in range(2):
            pltpu.make_async_copy(
                table_hbm.at[idx_eo.at[b, e]], rows_v.at[b, e], sem_g.at[b]
            ).wait()

    def start_wb(i, b):
        for e in range(2):
            pltpu.async_copy(rows_v.at[b, e], out_slice(i, e), sem_w.at[b])

    def wait_wb(i, b):
        for e in range(2):
            pltpu.make_async_copy(
                rows_v.at[b, e], out_slice(i, e), sem_w.at[b]
            ).wait()

    pltpu.sync_copy(x_hbm.at[pl.ds(base, per_w)], idx_all)

    split_idx(0, 0)
    start_gather(0, 0)

    def step(g, carry):
        for b in range(_NB):
            t = g * _NB + b

            @pl.when(t >= _NB)
            def _drain_slot():
                wait_wb(t - _NB, b)

            @pl.when(t >= 1)
            def _issue():
                split_idx(t, b)
                start_gather(t, b)

            c = t - _K
            cb = (b + _NB - _K) % _NB

            @pl.when(t >= _K)
            def _consume():
                wait_gather(c, cb)
                start_wb(c, cb)
        return carry

    lax.fori_loop(0, nchunk // _NB, step, 0)

    for j in range(nchunk - _K, nchunk):
        cb = j % _NB
        wait_gather(j, cb)
        start_wb(j, cb)
    for b in range(_NB):
        wait_wb(nchunk - _NB + b, b)


def kernel(x, table):
    b, h = x.shape
    n = b * h
    v, dim = table.shape
    xf = x.reshape(n)
    mesh = plsc.VectorSubcoreMesh(core_axis_name="c", subcore_axis_name="s")

    tt = jnp.swapaxes(table, 0, 1)
    ntail = v % 128
    tail = lax.slice(table, (v - ntail, 0), (v, dim)).reshape(ntail * dim // 128, 128)
    tab_lin = pl.kernel(
        _relayout_body,
        out_type=jax.ShapeDtypeStruct((_VP // 2, 128), table.dtype),
        mesh=mesh,
        scratch_types=[
            pltpu.VMEM((2, 64, 128), jnp.float32),
            pltpu.VMEM((2, 64, 128), jnp.float32),
            pltpu.VMEM((ntail * dim // 128, 128), jnp.float32),
            pltpu.SemaphoreType.DMA((2,)),
            pltpu.SemaphoreType.DMA((2,)),
        ],
        compiler_params=pltpu.CompilerParams(
            use_tc_tiling_on_sc=True, needs_layout_passes=False
        ),
    )(tt, tail)
    tab64 = tab_lin.reshape(_VP, 64)

    out = pl.kernel(
        _gather_body,
        out_type=jax.ShapeDtypeStruct((n // 2, 2 * dim), table.dtype),
        mesh=mesh,
        scratch_types=[
            pltpu.VMEM((n // _NW,), jnp.int32),
            pltpu.VMEM((_NB, 2, _CH // 2), jnp.int32),
            pltpu.VMEM((_NB, 2, _CH // 2, dim), jnp.float32),
            pltpu.SemaphoreType.DMA((_NB,)),
            pltpu.SemaphoreType.DMA((_NB,)),
        ],
        compiler_params=pltpu.CompilerParams(
            use_tc_tiling_on_sc=False, needs_layout_passes=False
        ),
    )(xf, tab64)
    return out.reshape(b, h, dim)


# speedup vs baseline: 1.1635x; 1.1635x over previous
import jax
import jax.numpy as jnp
from jax import lax
from jax.experimental import pallas as pl
from jax.experimental.pallas import tpu as pltpu
from jax.experimental.pallas import tpu_sc as plsc

_NC = 2
_NS = 16
_NW = _NC * _NS
_CH = 256
_NB = 5
_K = 3

_VP = 1000064


def _relayout_body(tt_hbm, tail_hbm, out_hbm, a_v, b_v, tail_v, sem_i, sem_o):
    nv = tt_hbm.shape[1]
    nblk = nv // 128
    wid = lax.axis_index("s") * _NC + lax.axis_index("c")
    per_w = (nblk + _NW - 1) // _NW
    iota = lax.iota(jnp.int32, 16)

    def blk(t):
        return t * _NW + wid

    def in_src(t):
        return tt_hbm.at[:, pl.ds(pl.multiple_of(blk(t) * 128, 128), 128)]

    def out_dst(t):
        return out_hbm.at[pl.ds(pl.multiple_of(blk(t) * 64, 64), 64)]

    @pl.when(blk(0) < nblk)
    def _prime():
        pltpu.async_copy(in_src(0), a_v.at[0], sem_i.at[0])

    def block(t, carry):
        s = t % 2

        @pl.when(blk(t + 1) < nblk)
        def _prefetch():
            pltpu.async_copy(in_src(t + 1), a_v.at[1 - s], sem_i.at[1 - s])

        @pl.when(blk(t) < nblk)
        def _wait_in():
            pltpu.make_async_copy(in_src(t), a_v.at[s], sem_i.at[s]).wait()

        @pl.when(jnp.logical_and(t >= 2, blk(t - 2) < nblk))
        def _drain():
            pltpu.make_async_copy(b_v.at[s], out_dst(t - 2), sem_o.at[s]).wait()

        plsc.subcore_barrier()

        @pl.when(blk(t) < nblk)
        def _transpose():
            @plsc.parallel_loop(0, 64, unroll=8)
            def _col_pair(vv):
                for par in range(2):
                    v = 2 * vv + par
                    for k in range(4):
                        vals = plsc.load_gather(
                            a_v.at[s],
                            [iota + 16 * k, jnp.full((16,), v, jnp.int32)],
                        )
                        b_v[s, vv, pl.ds(par * 64 + 16 * k, 16)] = vals

        plsc.subcore_barrier()

        @pl.when(blk(t) < nblk)
        def _start_out():
            pltpu.async_copy(b_v.at[s], out_dst(t), sem_o.at[s])

        return carry

    lax.fori_loop(0, per_w, block, 0)

    for d in range(2):
        t = per_w - 2 + d

        @pl.when(jnp.logical_and(t >= 0, blk(t) < nblk))
        def _final_drain():
            pltpu.make_async_copy(b_v.at[t % 2], out_dst(t), sem_o.at[t % 2]).wait()

    @pl.when(wid == 0)
    def _tail():
        pltpu.sync_copy(tail_hbm, tail_v)
        pltpu.sync_copy(tail_v, out_hbm.at[pl.ds((nblk * 128) // 2, tail_hbm.shape[0])])


def _gather_body_old(x_hbm, table_hbm, out_hbm, idx_all, rows_v, sem_g, sem_w):
    n = x_hbm.shape[0]
    per_w = n // _NW
    nchunk = per_w // _CH
    wid = lax.axis_index("s") * _NC + lax.axis_index("c")
    base = wid * per_w

    def idx_slice(i):
        return idx_all.at[pl.ds(i * _CH, _CH)]

    def out_slice(i):
        return out_hbm.at[pl.ds(base + i * _CH, _CH)]

    def chunk(i, carry):
        pltpu.async_copy(table_hbm.at[idx_slice(i)], rows_v.at[0], sem_g.at[0]).wait()
        pltpu.sync_copy(rows_v.at[0], out_slice(i))
        return carry

    pltpu.sync_copy(x_hbm.at[pl.ds(base, per_w)], idx_all)
    lax.fori_loop(0, nchunk, chunk, 0)


def _gather_body(x_hbm, table_hbm, out_hbm, idx_all, idx_eo, rows_v, sem_g, sem_w):
    n = x_hbm.shape[0]
    per_w = n // _NW
    nchunk = per_w // _CH
    half = _CH // 2
    wid = lax.axis_index("s") * _NC + lax.axis_index("c")
    base = wid * per_w
    iota = lax.iota(jnp.int32, 16)

    def split_idx(i, b):
        for e in range(2):
            for m in range(half // 16):
                src = i * _CH + e + 2 * (iota + 16 * m)
                idx_eo[b, e, pl.ds(16 * m, 16)] = plsc.load_gather(idx_all, [src])

    def out_slice(i, e):
        r0 = pl.multiple_of((base + i * _CH) // 2, half)
        return out_hbm.at[pl.ds(r0, half), pl.ds(e * 64, 64)]

    def start_gather(i, b):
        for e in range(2):
            pltpu.async_copy(
                table_hbm.at[idx_eo.at[b, e]], rows_v.at[b, e], sem_g.at[b]
            )

    def wait_gather(i, b):
        for e in range(2):
            pltpu.make_async_copy(
                table_hbm.at[idx_eo.at[b, e]], rows_v.at[b, e], sem_g.at[b]
            ).wait()

    def start_wb(i, b):
        for e in range(2):
            pltpu.async_copy(rows_v.at[b, e], out_slice(i, e), sem_w.at[b])

    def wait_wb(i, b):
        for e in range(2):
            pltpu.make_async_copy(
                rows_v.at[b, e], out_slice(i, e), sem_w.at[b]
            ).wait()

    pltpu.sync_copy(x_hbm.at[pl.ds(base, per_w)], idx_all)

    split_idx(0, 0)
    start_gather(0, 0)

    def step(g, carry):
        for b in range(_NB):
            t = g * _NB + b

            @pl.when(t >= _NB)
            def _drain_slot():
                wait_wb(t - _NB, b)

            @pl.when(t >= 1)
            def _issue():
                split_idx(t, b)
                start_gather(t, b)

            c = t - _K
            cb = (b + _NB - _K) % _NB

            @pl.when(t >= _K)
            def _consume():
                wait_gather(c, cb)
                start_wb(c, cb)
        return carry

    lax.fori_loop(0, nchunk // _NB, step, 0)

    for j in range(nchunk - _K, nchunk):
        cb = j % _NB
        wait_gather(j, cb)
        start_wb(j, cb)
    for b in range(_NB):
        wait_wb(nchunk - _NB + b, b)


def kernel(x, table):
    b, h = x.shape
    n = b * h
    v, dim = table.shape
    xf = x.reshape(n)
    mesh = plsc.VectorSubcoreMesh(core_axis_name="c", subcore_axis_name="s")

    tt = jnp.swapaxes(table, 0, 1)
    ntail = v % 128
    tail = lax.slice(table, (v - ntail, 0), (v, dim)).reshape(ntail * dim // 128, 128)
    use_sc_relayout = False
    if use_sc_relayout:
        tab_lin = pl.kernel(
            _relayout_body,
            out_type=jax.ShapeDtypeStruct((_VP // 2, 128), table.dtype),
            mesh=mesh,
            scratch_types=[
                pltpu.VMEM((2, 64, 128), jnp.float32),
                pltpu.VMEM((2, 64, 128), jnp.float32),
                pltpu.VMEM((ntail * dim // 128, 128), jnp.float32),
                pltpu.SemaphoreType.DMA((2,)),
                pltpu.SemaphoreType.DMA((2,)),
            ],
            compiler_params=pltpu.CompilerParams(
                use_tc_tiling_on_sc=True, needs_layout_passes=False
            ),
        )(tt, tail)
        tab64 = tab_lin.reshape(_VP, 64)
    else:
        tab64 = table

    out = pl.kernel(
        _gather_body,
        out_type=jax.ShapeDtypeStruct((n // 2, 2 * dim), table.dtype),
        mesh=mesh,
        scratch_types=[
            pltpu.VMEM((n // _NW,), jnp.int32),
            pltpu.VMEM((_NB, 2, _CH // 2), jnp.int32),
            pltpu.VMEM((_NB, 2, _CH // 2, dim), jnp.float32),
            pltpu.SemaphoreType.DMA((_NB,)),
            pltpu.SemaphoreType.DMA((_NB,)),
        ],
        compiler_params=pltpu.CompilerParams(
            use_tc_tiling_on_sc=False, needs_layout_passes=False
        ),
    )(xf, tab64)
    return out.reshape(b, h, dim)
